# baseline (speedup 1.0000x reference)
import functools

import jax
import jax.numpy as jnp
from jax import lax
from jax.experimental import pallas as pl
from jax.experimental.pallas import tpu as pltpu
from jax.experimental.pallas import tpu_sc as plsc

NC = 2
NS = 16
L = 16

SUB = 128
GSUB = 64
PAD_ROWS = 16
BLKROWS = 40


@functools.lru_cache(maxsize=None)
def _make_edge_prep(N, E):
    EROWS = E // SUB
    EROWS8 = -(-EROWS // 8) * 8
    NW = NC * NS
    RPP = -(-EROWS // NW // 8) * 8
    RPP_LAST = EROWS - (NW - 1) * RPP
    mesh = plsc.VectorSubcoreMesh(core_axis_name="c", subcore_axis_name="s")

    RPP_WLAST = EROWS8 - (NW - 1) * RPP

    @functools.partial(
        pl.kernel,
        out_type=(jax.ShapeDtypeStruct((2 * EROWS8, SUB), jnp.int32),
                  jax.ShapeDtypeStruct((EROWS8, SUB), jnp.int32)),
        mesh=mesh,
        scratch_types=[
            pltpu.VMEM((RPP * SUB,), jnp.int32),
            pltpu.VMEM((RPP, SUB), jnp.int32),
        ],
    )
    def prep(src_hbm, dst_hbm, srcm2_hbm, dstm_hbm, buf1d, buf2d):
        cid = lax.axis_index("c")
        sid = lax.axis_index("s")
        wid = sid * NC + cid

        ar16 = jnp.arange(L, dtype=jnp.int32)

        def do(rows, wrows):
            base = wid * RPP

            def repack(r, off):
                def row(rr, carry):
                    for k in range(SUB // L):
                        sl = pl.ds(k * L, L)
                        buf2d[rr, sl] = buf1d[pl.ds(rr * SUB + k * L, L)] + off
                    return carry
                lax.fori_loop(0, r, row, 0)

            def padfill(fn):
                for rr in range(rows, wrows):
                    for k in range(SUB // L):
                        buf2d[rr, pl.ds(k * L, L)] = fn(rr, k)

            pltpu.sync_copy(src_hbm.at[pl.ds(base * SUB, rows * SUB)],
                            buf1d.at[pl.ds(0, rows * SUB)])
            repack(rows, 0)
            padfill(lambda rr, k: ar16 + (rr * 8 + k) * L)
            pltpu.sync_copy(buf2d.at[pl.ds(0, wrows)],
                            srcm2_hbm.at[pl.ds(base, wrows)])
            repack(rows, jnp.int32(N))
            padfill(lambda rr, k: ar16 + (rr * 8 + k) * L + jnp.int32(N))
            pltpu.sync_copy(buf2d.at[pl.ds(0, wrows)],
                            srcm2_hbm.at[pl.ds(EROWS8 + base, wrows)])
            pltpu.sync_copy(dst_hbm.at[pl.ds(base * SUB, rows * SUB)],
                            buf1d.at[pl.ds(0, rows * SUB)])
            repack(rows, 0)
            padfill(lambda rr, k: ar16 + jnp.int32(N))
            pltpu.sync_copy(buf2d.at[pl.ds(0, wrows)],
                            dstm_hbm.at[pl.ds(base, wrows)])

        @pl.when(wid < NW - 1)
        def _():
            do(RPP, RPP)

        @pl.when(wid == NW - 1)
        def _():
            do(RPP_LAST, RPP_WLAST)

    return prep


@functools.lru_cache(maxsize=None)
def _make_spmm(N, E, HD):
    EROWS = E // SUB
    EROWS8 = -(-EROWS // 8) * 8
    ROWS_PAD = N + PAD_ROWS
    RPT = -(-EROWS8 // NS // 8) * 8
    RPT_LAST = EROWS8 - (NS - 1) * RPT
    RPI = -(-ROWS_PAD // NS // 8) * 8
    RPI_LAST = ROWS_PAD - (NS - 1) * RPI
    RPO = -(-N // NS // 8) * 8
    RPO_LAST = N - (NS - 1) * RPO

    def blocks_of(nrows):
        out, off = [], 0
        while off < nrows:
            nb = min(BLKROWS, nrows - off)
            out.append((off, nb))
            off += nb
        return tuple(out)

    BLOCKS_MAIN = blocks_of(RPT)
    BLOCKS_LAST = blocks_of(RPT_LAST)
    ARENA = 4 * GSUB
    mesh = plsc.VectorSubcoreMesh(core_axis_name="c", subcore_axis_name="s")

    @functools.partial(
        pl.kernel,
        out_type=jax.ShapeDtypeStruct((NC * N, HD), jnp.float32),
        mesh=mesh,
        scratch_types=[
            pltpu.VMEM((BLKROWS, SUB), jnp.int32),
            pltpu.VMEM((BLKROWS, SUB), jnp.int32),
            pltpu.VMEM((ARENA, HD), jnp.float32),
            pltpu.VMEM_SHARED((ROWS_PAD, HD), jnp.float32),
            pltpu.SemaphoreType.DMA,
            pltpu.SemaphoreType.DMA,
            pltpu.SemaphoreType.DMA,
            pltpu.SemaphoreType.DMA,
            pltpu.SemaphoreType.DMA,
            pltpu.SemaphoreType.DMA,
        ],
    )
    def spmm(h_hbm, srcm_hbm, dstm_hbm, out_hbm,
             src_v, dst_v, arena, acc_s,
             qsem0, qsem1, qsem2, qsem3, ssem0, ssem1):
        cid = lax.axis_index("c")
        sid = lax.axis_index("s")
        cN = cid * N
        zeros16 = jnp.zeros((L,), jnp.float32)

        def zrow(r, carry):
            for k in range(HD // L):
                arena[r, pl.ds(k * L, L)] = zeros16
            return carry

        lax.fori_loop(0, ARENA, zrow, 0)

        def zinit(base, nrows):
            off = 0
            while off < nrows:
                nn = min(ARENA, nrows - off)
                pltpu.sync_copy(arena.at[pl.ds(0, nn)],
                                acc_s.at[pl.ds(base + off, nn)])
                off += nn

        @pl.when(sid < NS - 1)
        def _():
            zinit(sid * RPI, RPI)

        @pl.when(sid == NS - 1)
        def _():
            zinit((NS - 1) * RPI, RPI_LAST)

        plsc.subcore_barrier()

        qsems = (qsem0, qsem1, qsem2, qsem3)

        def gath(r, h, issue, par):
            q = 2 * par + h
            d = (pltpu.async_copy if issue else pltpu.make_async_copy)(
                h_hbm.at[src_v.at[r, pl.ds(h * GSUB, GSUB)]],
                arena.at[pl.ds(q * GSUB, GSUB)], qsems[q])
            if not issue:
                d.wait()

        def scat(k, issue, half):
            sem = ssem0 if half == 0 else ssem1
            if issue:
                pltpu.async_copy(arena.at[pl.ds(half * SUB, SUB)],
                                 acc_s.at[dst_v.at[k]], sem, add=True)
            else:
                pltpu.make_async_copy(arena.at[pl.ds(half * SUB, SUB)],
                                      acc_s.at[dst_v.at[k]], sem).wait()

        def run_block(rb, nrows):
            pltpu.sync_copy(srcm_hbm.at[pl.ds(cid * EROWS8 + rb, nrows)],
                            src_v.at[pl.ds(0, nrows)])
            pltpu.sync_copy(dstm_hbm.at[pl.ds(rb, nrows)],
                            dst_v.at[pl.ds(0, nrows)])

            gath(0, 0, True, 0)
            gath(0, 1, True, 0)
            gath(1, 0, True, 1)
            gath(1, 1, True, 1)

            def step_pair(i, carry):
                kA = 2 * i
                gath(kA, 0, False, 0)
                gath(kA, 1, False, 0)

                @pl.when(i > 0)
                def _():
                    scat(kA - 1, False, 1)
                    gath(kA + 1, 0, True, 1)
                    gath(kA + 1, 1, True, 1)

                scat(kA, True, 0)
                gath(kA + 1, 0, False, 1)
                gath(kA + 1, 1, False, 1)
                scat(kA, False, 0)

                @pl.when(i < nrows // 2 - 1)
                def _():
                    gath(kA + 2, 0, True, 0)
                    gath(kA + 2, 1, True, 0)

                scat(kA + 1, True, 1)
                return carry

            lax.fori_loop(0, nrows // 2, step_pair, 0)
            scat(nrows - 1, False, 1)

        @pl.when(sid < NS - 1)
        def _():
            for off, nb in BLOCKS_MAIN:
                run_block(sid * RPT + off, nb)

        @pl.when(sid == NS - 1)
        def _():
            for off, nb in BLOCKS_LAST:
                run_block((NS - 1) * RPT + off, nb)

        plsc.subcore_barrier()

        @pl.when(sid < NS - 1)
        def _():
            pltpu.sync_copy(acc_s.at[pl.ds(sid * RPO, RPO)],
                            out_hbm.at[pl.ds(cN + sid * RPO, RPO)])

        @pl.when(sid == NS - 1)
        def _():
            pltpu.sync_copy(acc_s.at[pl.ds((NS - 1) * RPO, RPO_LAST)],
                            out_hbm.at[pl.ds(cN + (NS - 1) * RPO, RPO_LAST)])

    return spmm


def _make_gemm(N, Hout, relu, BM=2000):
    NB = N // BM
    HB = Hout // 128

    def kern(scale_ref, s0, s1, h0, h1, w_ref, b_ref, o_ref):
        sc = scale_ref[0, 0]
        a0 = h0[...] * sc + s0[...]
        a1 = h1[...] * sc + s1[...]
        z = (jnp.dot(a0, w_ref[:128, :], preferred_element_type=jnp.float32)
             + jnp.dot(a1, w_ref[128:, :], preferred_element_type=jnp.float32)
             + b_ref[...])
        if relu:
            z = jnp.maximum(z, 0.0)
        o_ref[...] = z

    return pl.pallas_call(
        kern,
        grid=(NB, HB),
        in_specs=[
            pl.BlockSpec(memory_space=pltpu.SMEM),
            pl.BlockSpec((BM, 128), lambda i, j: (i, 0)),
            pl.BlockSpec((BM, 128), lambda i, j: (i + NB, 0)),
            pl.BlockSpec((BM, 128), lambda i, j: (i, 0)),
            pl.BlockSpec((BM, 128), lambda i, j: (i + NB, 0)),
            pl.BlockSpec((256, 128), lambda i, j: (0, j)),
            pl.BlockSpec((1, 128), lambda i, j: (0, j)),
        ],
        out_specs=pl.BlockSpec((BM, 128), lambda i, j: (j * NB + i, 0)),
        out_shape=jax.ShapeDtypeStruct((HB * N, 128), jnp.float32),
        compiler_params=pltpu.CompilerParams(
            dimension_semantics=("arbitrary", "arbitrary")),
    )


def kernel(x, edge_index, eps, W1, b1, W2, b2, W3, b3):
    N, D = x.shape
    E = edge_index.shape[1]
    HD = D // 2

    x2 = jnp.concatenate([x[:, :HD], x[:, HD:]], axis=0)

    ei = edge_index.astype(jnp.int32)
    srcm, dstm = _make_edge_prep(N, E)(ei[0], ei[1])

    spmm = _make_spmm(N, E, HD)
    gemm_h1 = _make_gemm(N, W1.shape[1], relu=True)
    gemm_h2 = _make_gemm(N, W2.shape[1], relu=True)
    gemm_z = _make_gemm(N, W3.shape[1], relu=False)

    scales = (1.0 + eps).reshape(-1, 1, 1)

    s = spmm(x2, srcm, dstm)
    h = gemm_h1(scales[0], s, s, x2, x2, W1, b1.reshape(1, -1))
    s = spmm(h, srcm, dstm)
    h = gemm_h2(scales[1], s, s, h, h, W2, b2.reshape(1, -1))
    s = spmm(h, srcm, dstm)
    z = gemm_z(scales[2], s, s, h, h, W3, b3.reshape(1, -1))
    return z


# speedup vs baseline: 1.0035x; 1.0035x over previous
import functools

import jax
import jax.numpy as jnp
from jax import lax
from jax.experimental import pallas as pl
from jax.experimental.pallas import tpu as pltpu
from jax.experimental.pallas import tpu_sc as plsc

NC = 2
NS = 16
L = 16

SUB = 128
GSUB = 64
BLKROWS = 40


@functools.lru_cache(maxsize=None)
def _make_spmm(N, E, HD):
    EROWS = E // SUB
    RPT = -(-EROWS // NS // 8) * 8
    RPT_LAST = EROWS - (NS - 1) * RPT
    RPO = -(-N // NS // 8) * 8
    RPO_LAST = N - (NS - 1) * RPO

    def blocks_of(nrows):
        out, off = [], 0
        while off < nrows:
            nb = min(BLKROWS, nrows - off)
            out.append((off, nb))
            off += nb
        return tuple(out)

    BLOCKS_MAIN = blocks_of(RPT)
    BLOCKS_LAST = blocks_of(RPT_LAST)
    ARENA = 4 * GSUB
    mesh = plsc.VectorSubcoreMesh(core_axis_name="c", subcore_axis_name="s")

    @functools.partial(
        pl.kernel,
        out_type=jax.ShapeDtypeStruct((NC * N, HD), jnp.float32),
        mesh=mesh,
        scratch_types=[
            pltpu.VMEM((BLKROWS, SUB), jnp.int32),
            pltpu.VMEM((BLKROWS, SUB), jnp.int32),
            pltpu.VMEM((ARENA, HD), jnp.float32),
            pltpu.VMEM_SHARED((N, HD), jnp.float32),
            pltpu.SemaphoreType.DMA,
            pltpu.SemaphoreType.DMA,
            pltpu.SemaphoreType.DMA,
            pltpu.SemaphoreType.DMA,
            pltpu.SemaphoreType.DMA,
            pltpu.SemaphoreType.DMA,
        ],
    )
    def spmm(h_hbm, srcm_hbm, dstm_hbm, out_hbm,
             src_v, dst_v, arena, acc_s,
             qsem0, qsem1, qsem2, qsem3, ssem0, ssem1):
        cid = lax.axis_index("c")
        sid = lax.axis_index("s")
        cN = cid * N
        zeros16 = jnp.zeros((L,), jnp.float32)

        def zrow(r, carry):
            for k in range(HD // L):
                arena[r, pl.ds(k * L, L)] = zeros16
            return carry

        lax.fori_loop(0, ARENA, zrow, 0)

        def zinit(base, nrows):
            off = 0
            while off < nrows:
                nn = min(ARENA, nrows - off)
                pltpu.sync_copy(arena.at[pl.ds(0, nn)],
                                acc_s.at[pl.ds(base + off, nn)])
                off += nn

        @pl.when(sid < NS - 1)
        def _():
            zinit(sid * RPO, RPO)

        @pl.when(sid == NS - 1)
        def _():
            zinit((NS - 1) * RPO, RPO_LAST)

        plsc.subcore_barrier()

        qsems = (qsem0, qsem1, qsem2, qsem3)

        def gath(r, h, issue, par):
            q = 2 * par + h
            d = (pltpu.async_copy if issue else pltpu.make_async_copy)(
                h_hbm.at[src_v.at[r, pl.ds(h * GSUB, GSUB)]],
                arena.at[pl.ds(q * GSUB, GSUB)], qsems[q])
            if not issue:
                d.wait()

        def scat(k, issue, half):
            sem = ssem0 if half == 0 else ssem1
            if issue:
                pltpu.async_copy(arena.at[pl.ds(half * SUB, SUB)],
                                 acc_s.at[dst_v.at[k]], sem, add=True)
            else:
                pltpu.make_async_copy(arena.at[pl.ds(half * SUB, SUB)],
                                      acc_s.at[dst_v.at[k]], sem).wait()

        def run_block(rb, nrows):
            pltpu.sync_copy(srcm_hbm.at[pl.ds(rb, nrows)],
                            src_v.at[pl.ds(0, nrows)])
            pltpu.sync_copy(dstm_hbm.at[pl.ds(rb, nrows)],
                            dst_v.at[pl.ds(0, nrows)])

            def addrow(r, carry):
                for k in range(SUB // L):
                    sl = pl.ds(k * L, L)
                    src_v[r, sl] = src_v[r, sl] + cN
                return carry

            lax.fori_loop(0, nrows, addrow, 0)

            gath(0, 0, True, 0)
            gath(0, 1, True, 0)
            gath(1, 0, True, 1)
            gath(1, 1, True, 1)

            def step_pair(i, carry):
                kA = 2 * i
                gath(kA, 0, False, 0)
                gath(kA, 1, False, 0)

                @pl.when(i > 0)
                def _():
                    scat(kA - 1, False, 1)
                    gath(kA + 1, 0, True, 1)
                    gath(kA + 1, 1, True, 1)

                scat(kA, True, 0)
                gath(kA + 1, 0, False, 1)
                gath(kA + 1, 1, False, 1)
                scat(kA, False, 0)

                @pl.when(i < nrows // 2 - 1)
                def _():
                    gath(kA + 2, 0, True, 0)
                    gath(kA + 2, 1, True, 0)

                scat(kA + 1, True, 1)
                return carry

            lax.fori_loop(0, nrows // 2, step_pair, 0)
            scat(nrows - 1, False, 1)

        @pl.when(sid < NS - 1)
        def _():
            for off, nb in BLOCKS_MAIN:
                run_block(sid * RPT + off, nb)

        @pl.when(sid == NS - 1)
        def _():
            for off, nb in BLOCKS_LAST:
                run_block((NS - 1) * RPT + off, nb)

        plsc.subcore_barrier()

        @pl.when(sid < NS - 1)
        def _():
            pltpu.sync_copy(acc_s.at[pl.ds(sid * RPO, RPO)],
                            out_hbm.at[pl.ds(cN + sid * RPO, RPO)])

        @pl.when(sid == NS - 1)
        def _():
            pltpu.sync_copy(acc_s.at[pl.ds((NS - 1) * RPO, RPO_LAST)],
                            out_hbm.at[pl.ds(cN + (NS - 1) * RPO, RPO_LAST)])

    return spmm


def _make_split(N, D, BM=2000):
    NB = N // BM

    def kern(x_ref, o_ref):
        o_ref[...] = x_ref[...]

    return pl.pallas_call(
        kern,
        grid=(NB, 2),
        in_specs=[pl.BlockSpec((BM, D // 2), lambda i, j: (i, j))],
        out_specs=pl.BlockSpec((BM, D // 2), lambda i, j: (j * NB + i, 0)),
        out_shape=jax.ShapeDtypeStruct((2 * N, D // 2), jnp.float32),
        compiler_params=pltpu.CompilerParams(
            dimension_semantics=("arbitrary", "arbitrary")),
    )


def _make_gemm(N, Hout, relu, BM=2000):
    NB = N // BM
    HB = Hout // 128

    def kern(scale_ref, s0, s1, h0, h1, w_ref, b_ref, o_ref):
        sc = scale_ref[0, 0]
        a0 = h0[...] * sc + s0[...]
        a1 = h1[...] * sc + s1[...]
        z = (jnp.dot(a0, w_ref[:128, :], preferred_element_type=jnp.float32)
             + jnp.dot(a1, w_ref[128:, :], preferred_element_type=jnp.float32)
             + b_ref[...])
        if relu:
            z = jnp.maximum(z, 0.0)
        o_ref[...] = z

    return pl.pallas_call(
        kern,
        grid=(NB, HB),
        in_specs=[
            pl.BlockSpec(memory_space=pltpu.SMEM),
            pl.BlockSpec((BM, 128), lambda i, j: (i, 0)),
            pl.BlockSpec((BM, 128), lambda i, j: (i + NB, 0)),
            pl.BlockSpec((BM, 128), lambda i, j: (i, 0)),
            pl.BlockSpec((BM, 128), lambda i, j: (i + NB, 0)),
            pl.BlockSpec((256, 128), lambda i, j: (0, j)),
            pl.BlockSpec((1, 128), lambda i, j: (0, j)),
        ],
        out_specs=pl.BlockSpec((BM, 128), lambda i, j: (j * NB + i, 0)),
        out_shape=jax.ShapeDtypeStruct((HB * N, 128), jnp.float32),
        compiler_params=pltpu.CompilerParams(
            dimension_semantics=("arbitrary", "arbitrary")),
    )


def kernel(x, edge_index, eps, W1, b1, W2, b2, W3, b3):
    N, D = x.shape
    E = edge_index.shape[1]
    HD = D // 2

    x2 = _make_split(N, D)(x)

    srcm = edge_index[0].astype(jnp.int32).reshape(E // SUB, SUB)
    dstm = edge_index[1].astype(jnp.int32).reshape(E // SUB, SUB)

    spmm = _make_spmm(N, E, HD)
    gemm_h1 = _make_gemm(N, W1.shape[1], relu=True)
    gemm_h2 = _make_gemm(N, W2.shape[1], relu=True)
    gemm_z = _make_gemm(N, W3.shape[1], relu=False)

    scales = (1.0 + eps).reshape(-1, 1, 1)

    s = spmm(x2, srcm, dstm)
    h = gemm_h1(scales[0], s, s, x2, x2, W1, b1.reshape(1, -1))
    s = spmm(h, srcm, dstm)
    h = gemm_h2(scales[1], s, s, h, h, W2, b2.reshape(1, -1))
    s = spmm(h, srcm, dstm)
    z = gemm_z(scales[2], s, s, h, h, W3, b3.reshape(1, -1))
    return z


# speedup vs baseline: 1.0040x; 1.0006x over previous
import functools

import jax
import jax.numpy as jnp
from jax import lax
from jax.experimental import pallas as pl
from jax.experimental.pallas import tpu as pltpu
from jax.experimental.pallas import tpu_sc as plsc

NC = 2
NS = 16
L = 16

SUB = 128
GSUB = 64
BLKROWS = 40


@functools.lru_cache(maxsize=None)
def _make_spmm(N, E, HD):
    EROWS = E // SUB
    RPT = -(-EROWS // NS // 8) * 8
    RPT_LAST = EROWS - (NS - 1) * RPT
    RPO = -(-N // NS // 8) * 8
    RPO_LAST = N - (NS - 1) * RPO

    def blocks_of(nrows):
        out, off = [], 0
        while off < nrows:
            nb = min(BLKROWS, nrows - off)
            out.append((off, nb))
            off += nb
        return tuple(out)

    BLOCKS_MAIN = blocks_of(RPT)
    BLOCKS_LAST = blocks_of(RPT_LAST)
    ARENA = 4 * GSUB
    mesh = plsc.VectorSubcoreMesh(core_axis_name="c", subcore_axis_name="s")

    @functools.partial(
        pl.kernel,
        out_type=jax.ShapeDtypeStruct((NC * N, HD), jnp.float32),
        mesh=mesh,
        scratch_types=[
            pltpu.VMEM((BLKROWS, SUB), jnp.int32),
            pltpu.VMEM((BLKROWS, SUB), jnp.int32),
            pltpu.VMEM((ARENA, HD), jnp.float32),
            pltpu.VMEM_SHARED((N, HD), jnp.float32),
            pltpu.SemaphoreType.DMA,
            pltpu.SemaphoreType.DMA,
            pltpu.SemaphoreType.DMA,
            pltpu.SemaphoreType.DMA,
            pltpu.SemaphoreType.DMA,
            pltpu.SemaphoreType.DMA,
        ],
    )
    def spmm(h_hbm, srcm_hbm, dstm_hbm, out_hbm,
             src_v, dst_v, arena, acc_s,
             qsem0, qsem1, qsem2, qsem3, ssem0, ssem1):
        cid = lax.axis_index("c")
        sid = lax.axis_index("s")
        cN = cid * N
        zeros16 = jnp.zeros((L,), jnp.float32)

        def zrow(r, carry):
            for k in range(HD // L):
                arena[r, pl.ds(k * L, L)] = zeros16
            return carry

        lax.fori_loop(0, ARENA, zrow, 0)

        def zinit(base, nrows):
            off = 0
            while off < nrows:
                nn = min(ARENA, nrows - off)
                pltpu.sync_copy(arena.at[pl.ds(0, nn)],
                                acc_s.at[pl.ds(base + off, nn)])
                off += nn

        @pl.when(sid < NS - 1)
        def _():
            zinit(sid * RPO, RPO)

        @pl.when(sid == NS - 1)
        def _():
            zinit((NS - 1) * RPO, RPO_LAST)

        plsc.subcore_barrier()

        qsems = (qsem0, qsem1, qsem2, qsem3)

        def gath(r, h, issue, par):
            q = 2 * par + h
            d = (pltpu.async_copy if issue else pltpu.make_async_copy)(
                h_hbm.at[src_v.at[r, pl.ds(h * GSUB, GSUB)]],
                arena.at[pl.ds(q * GSUB, GSUB)], qsems[q])
            if not issue:
                d.wait()

        def scat(k, issue, half):
            sem = ssem0 if half == 0 else ssem1
            if issue:
                pltpu.async_copy(arena.at[pl.ds(half * SUB, SUB)],
                                 acc_s.at[dst_v.at[k]], sem, add=True)
            else:
                pltpu.make_async_copy(arena.at[pl.ds(half * SUB, SUB)],
                                      acc_s.at[dst_v.at[k]], sem).wait()

        def run_block(rb, nrows):
            pltpu.sync_copy(srcm_hbm.at[pl.ds(rb, nrows)],
                            src_v.at[pl.ds(0, nrows)])
            pltpu.sync_copy(dstm_hbm.at[pl.ds(rb, nrows)],
                            dst_v.at[pl.ds(0, nrows)])

            def addrow(r, carry):
                for k in range(SUB // L):
                    sl = pl.ds(k * L, L)
                    src_v[r, sl] = src_v[r, sl] + cN
                return carry

            lax.fori_loop(0, 2, addrow, 0)
            gath(0, 0, True, 0)
            gath(0, 1, True, 0)
            gath(1, 0, True, 1)
            gath(1, 1, True, 1)
            lax.fori_loop(2, nrows, addrow, 0)

            def step_pair(i, carry):
                kA = 2 * i
                gath(kA, 0, False, 0)
                gath(kA, 1, False, 0)

                @pl.when(i > 0)
                def _():
                    scat(kA - 1, False, 1)
                    gath(kA + 1, 0, True, 1)
                    gath(kA + 1, 1, True, 1)

                scat(kA, True, 0)
                gath(kA + 1, 0, False, 1)
                gath(kA + 1, 1, False, 1)
                scat(kA, False, 0)

                @pl.when(i < nrows // 2 - 1)
                def _():
                    gath(kA + 2, 0, True, 0)
                    gath(kA + 2, 1, True, 0)

                scat(kA + 1, True, 1)
                return carry

            lax.fori_loop(0, nrows // 2, step_pair, 0)
            scat(nrows - 1, False, 1)

        @pl.when(sid < NS - 1)
        def _():
            for off, nb in BLOCKS_MAIN:
                run_block(sid * RPT + off, nb)

        @pl.when(sid == NS - 1)
        def _():
            for off, nb in BLOCKS_LAST:
                run_block((NS - 1) * RPT + off, nb)

        plsc.subcore_barrier()

        @pl.when(sid < NS - 1)
        def _():
            pltpu.sync_copy(acc_s.at[pl.ds(sid * RPO, RPO)],
                            out_hbm.at[pl.ds(cN + sid * RPO, RPO)])

        @pl.when(sid == NS - 1)
        def _():
            pltpu.sync_copy(acc_s.at[pl.ds((NS - 1) * RPO, RPO_LAST)],
                            out_hbm.at[pl.ds(cN + (NS - 1) * RPO, RPO_LAST)])

    return spmm


def _make_split(N, D, BM=2000):
    NB = N // BM

    def kern(x_ref, o_ref):
        o_ref[...] = x_ref[...]

    return pl.pallas_call(
        kern,
        grid=(NB, 2),
        in_specs=[pl.BlockSpec((BM, D // 2), lambda i, j: (i, j))],
        out_specs=pl.BlockSpec((BM, D // 2), lambda i, j: (j * NB + i, 0)),
        out_shape=jax.ShapeDtypeStruct((2 * N, D // 2), jnp.float32),
        compiler_params=pltpu.CompilerParams(
            dimension_semantics=("arbitrary", "arbitrary")),
    )


def _make_gemm(N, Hout, relu, BM=2000):
    NB = N // BM
    HB = Hout // 128

    def kern(scale_ref, s0, s1, h0, h1, w_ref, b_ref, o_ref):
        sc = scale_ref[0, 0]
        a0 = h0[...] * sc + s0[...]
        a1 = h1[...] * sc + s1[...]
        z = (jnp.dot(a0, w_ref[:128, :], preferred_element_type=jnp.float32)
             + jnp.dot(a1, w_ref[128:, :], preferred_element_type=jnp.float32)
             + b_ref[...])
        if relu:
            z = jnp.maximum(z, 0.0)
        o_ref[...] = z

    return pl.pallas_call(
        kern,
        grid=(NB, HB),
        in_specs=[
            pl.BlockSpec(memory_space=pltpu.SMEM),
            pl.BlockSpec((BM, 128), lambda i, j: (i, 0)),
            pl.BlockSpec((BM, 128), lambda i, j: (i + NB, 0)),
            pl.BlockSpec((BM, 128), lambda i, j: (i, 0)),
            pl.BlockSpec((BM, 128), lambda i, j: (i + NB, 0)),
            pl.BlockSpec((256, 128), lambda i, j: (0, j)),
            pl.BlockSpec((1, 128), lambda i, j: (0, j)),
        ],
        out_specs=pl.BlockSpec((BM, 128), lambda i, j: (j * NB + i, 0)),
        out_shape=jax.ShapeDtypeStruct((HB * N, 128), jnp.float32),
        compiler_params=pltpu.CompilerParams(
            dimension_semantics=("arbitrary", "arbitrary")),
    )


def kernel(x, edge_index, eps, W1, b1, W2, b2, W3, b3):
    N, D = x.shape
    E = edge_index.shape[1]
    HD = D // 2

    x2 = _make_split(N, D)(x)

    srcm = edge_index[0].astype(jnp.int32).reshape(E // SUB, SUB)
    dstm = edge_index[1].astype(jnp.int32).reshape(E // SUB, SUB)

    spmm = _make_spmm(N, E, HD)
    gemm_h1 = _make_gemm(N, W1.shape[1], relu=True)
    gemm_h2 = _make_gemm(N, W2.shape[1], relu=True)
    gemm_z = _make_gemm(N, W3.shape[1], relu=False)

    scales = (1.0 + eps).reshape(-1, 1, 1)

    s = spmm(x2, srcm, dstm)
    h = gemm_h1(scales[0], s, s, x2, x2, W1, b1.reshape(1, -1))
    s = spmm(h, srcm, dstm)
    h = gemm_h2(scales[1], s, s, h, h, W2, b2.reshape(1, -1))
    s = spmm(h, srcm, dstm)
    z = gemm_z(scales[2], s, s, h, h, W3, b3.reshape(1, -1))
    return z
